# Initial kernel scaffold; baseline (speedup 1.0000x reference)
import jax
import jax.numpy as jnp
from jax.experimental import pallas as pl


def kernel(inputs, edge_index0, edge_index1, W0, b0, W1, b1, att0, att1, fc_W, fc_b, h_bias):
    raise NotImplementedError("write your pallas kernel here")


# speedup vs baseline: 3.8969x; 3.8969x over previous
import functools

import jax
import jax.numpy as jnp
from jax import lax
from jax.experimental import pallas as pl
from jax.experimental.pallas import tpu as pltpu
from jax.experimental.pallas import tpu_sc as plsc

N = 10000
E = 320000
D = 128
NC = 2
NS = 16
CL = 128
CH = (E // NS + CL - 1) // CL
EPT = CH * CL
EP = NS * EPT
NP = 10240
RPT = NP // NS

_mesh = plsc.VectorSubcoreMesh(core_axis_name="c", subcore_axis_name="s")


@functools.partial(
    pl.kernel,
    mesh=_mesh,
    out_type=jax.ShapeDtypeStruct((NC, 2, NP), jnp.float32),
    scratch_types=[
        pltpu.VMEM((CH, CL), jnp.int32),
        pltpu.VMEM((CL,), jnp.float32),
        pltpu.VMEM((RPT,), jnp.float32),
        pltpu.VMEM_SHARED((NP,), jnp.float32),
        pltpu.VMEM_SHARED((NP,), jnp.float32),
    ],
)
def _sc_degrees(degsrc_hbm, dst_hbm, out_hbm, idx_v, ones_v, stage_v,
                degs_sh, degd_sh):
    c = lax.axis_index("c")
    s = lax.axis_index("s")

    def _fill(i, _):
        stage_v[pl.ds(i * 16, 16)] = jnp.zeros((16,), jnp.float32)
        return 0
    lax.fori_loop(0, RPT // 16, _fill, 0)
    def _fill1(i, _):
        ones_v[pl.ds(i * 16, 16)] = jnp.ones((16,), jnp.float32)
        return 0
    lax.fori_loop(0, CL // 16, _fill1, 0)

    pltpu.sync_copy(stage_v, degs_sh.at[pl.ds(s * RPT, RPT)])
    pltpu.sync_copy(stage_v, degd_sh.at[pl.ds(s * RPT, RPT)])
    plsc.subcore_barrier()

    pltpu.sync_copy(degsrc_hbm.at[c, s], idx_v)
    def _cnt_s(j, _):
        pltpu.sync_copy(ones_v, degs_sh.at[idx_v.at[j]], add=True)
        return 0
    lax.fori_loop(0, CH, _cnt_s, 0)

    pltpu.sync_copy(dst_hbm.at[c, s], idx_v)
    def _cnt_d(j, _):
        pltpu.sync_copy(ones_v, degd_sh.at[idx_v.at[j]], add=True)
        return 0
    lax.fori_loop(0, CH, _cnt_d, 0)

    plsc.subcore_barrier()

    pltpu.sync_copy(degs_sh.at[pl.ds(s * RPT, RPT)], stage_v)
    pltpu.sync_copy(stage_v, out_hbm.at[c, 0, pl.ds(s * RPT, RPT)])
    pltpu.sync_copy(degd_sh.at[pl.ds(s * RPT, RPT)], stage_v)
    pltpu.sync_copy(stage_v, out_hbm.at[c, 1, pl.ds(s * RPT, RPT)])


DH = D // 2


@functools.partial(
    pl.kernel,
    mesh=_mesh,
    out_type=jax.ShapeDtypeStruct((2, NC, NP, DH), jnp.float32),
    compiler_params=pltpu.CompilerParams(use_tc_tiling_on_sc=False),
    scratch_types=[
        pltpu.VMEM((CH, CL), jnp.int32),
        pltpu.VMEM((CH, CL), jnp.int32),
        pltpu.VMEM((CL, DH), jnp.float32),
        pltpu.VMEM((CL, DH), jnp.float32),
        pltpu.VMEM_SHARED((NP, DH), jnp.float32),
        pltpu.SemaphoreType.DMA,
        pltpu.SemaphoreType.DMA,
    ],
)
def _sc_aggregate(h_hbm, src_hbm, dst_hbm, out_hbm, src_v, dst_v,
                  buf0, buf1, agg_sh, sem0, sem1):
    c = lax.axis_index("c")
    s = lax.axis_index("s")

    pltpu.sync_copy(dst_hbm.at[c, s], dst_v)

    for p in range(2):
        def _fill(i, _):
            r = i // (DH // 16)
            k = i % (DH // 16)
            buf0[r, pl.ds(k * 16, 16)] = jnp.zeros((16,), jnp.float32)
            return 0
        lax.fori_loop(0, CL * (DH // 16), _fill, 0)
        for b in range(RPT // CL):
            pltpu.sync_copy(buf0, agg_sh.at[pl.ds(s * RPT + b * CL, CL), :])

        pltpu.sync_copy(src_hbm.at[p, c, s], src_v)
        plsc.subcore_barrier()

        def _step(j, _):
            pltpu.async_copy(h_hbm.at[src_v.at[j]], buf0, sem0).wait()
            pltpu.sync_copy(buf0, agg_sh.at[dst_v.at[j]], add=True)
            return 0

        lax.fori_loop(0, CH, _step, 0)
        plsc.subcore_barrier()

        for b in range(RPT // CL):
            r0 = s * RPT + b * CL
            pltpu.sync_copy(agg_sh.at[pl.ds(r0, CL), :], buf0)
            pltpu.sync_copy(buf0, out_hbm.at[p, c, pl.ds(r0, CL), :])
        plsc.subcore_barrier()


def _tc_scale_body(deg_ref, x_ref, h_ref, rin_ref):
    g = pl.program_id(0)
    i = pl.program_id(1)
    blk = x_ref.shape[0]
    od = deg_ref[g, 0, pl.ds(i * blk, blk)]
    scale = lax.rsqrt(jnp.maximum(od, 1.0))
    h = x_ref[...] * scale[:, None]
    h_ref[0] = h[:, :D // 2]
    h_ref[1] = h[:, D // 2:]
    ind = deg_ref[g, 1, :]
    rin_ref[0, 0] = lax.rsqrt(jnp.maximum(ind, 1.0))


def _tc_scale(deg, x, blk=512):
    nb = NP // blk
    return pl.pallas_call(
        _tc_scale_body,
        grid=(NC, nb),
        in_specs=[
            pl.BlockSpec((NC, 2, NP), lambda g, i: (0, 0, 0)),
            pl.BlockSpec((blk, D), lambda g, i: (i, 0)),
        ],
        out_specs=[
            pl.BlockSpec((2, blk, D // 2), lambda g, i: (0, g * nb + i, 0)),
            pl.BlockSpec((1, 1, NP), lambda g, i: (g, 0, 0)),
        ],
        out_shape=[
            jax.ShapeDtypeStruct((2, NC * NP, D // 2), jnp.float32),
            jax.ShapeDtypeStruct((NC, 1, NP), jnp.float32),
        ],
    )(deg, x)


def _tc_feat_body(agg_ref, rin_ref, w_ref, b_ref, fcw_ref, fcb_ref,
                  out_ref, sp_ref, acc_ref):
    i = pl.program_id(0)
    blk = agg_ref.shape[1]

    @pl.when(i == 0)
    def _():
        acc_ref[...] = jnp.zeros_like(acc_ref)

    row0 = i * blk
    rows = row0 + lax.broadcasted_iota(jnp.int32, (blk, 1), 0)
    mask = rows < N

    for g in range(NC):
        scale = rin_ref[g, 0, pl.ds(row0, blk)][:, None]
        a = agg_ref[g] * scale
        out = jnp.dot(a, w_ref[g], preferred_element_type=jnp.float32)
        out = out + b_ref[g]
        out_ref[g] = out
        u = jnp.tanh(
            jax.lax.dot_general(out, fcw_ref[0],
                                (((1,), (1,)), ((), ())),
                                preferred_element_type=jnp.float32)
            + fcb_ref[0])
        u = jnp.where(mask, u, 0.0)
        acc_ref[g, :] += jnp.sum(u, axis=0)

    sp_ref[...] = acc_ref[...] * (1.0 / N)


def _tc_feat(agg, rin, w, b, fcw, fcb, blk=512):
    nb = NP // blk
    return pl.pallas_call(
        _tc_feat_body,
        grid=(nb,),
        in_specs=[
            pl.BlockSpec((NC, blk, D), lambda i: (0, i, 0)),
            pl.BlockSpec((NC, 1, NP), lambda i: (0, 0, 0)),
            pl.BlockSpec((NC, D, D), lambda i: (0, 0, 0)),
            pl.BlockSpec((NC, 1, D), lambda i: (0, 0, 0)),
            pl.BlockSpec((1, D, D), lambda i: (0, 0, 0)),
            pl.BlockSpec((1, D), lambda i: (0, 0)),
        ],
        out_specs=[
            pl.BlockSpec((NC, blk, D), lambda i: (0, i, 0)),
            pl.BlockSpec((NC, D), lambda i: (0, 0)),
        ],
        out_shape=[
            jax.ShapeDtypeStruct((NC, NP, D), jnp.float32),
            jax.ShapeDtypeStruct((NC, D), jnp.float32),
        ],
        scratch_shapes=[pltpu.VMEM((NC, D), jnp.float32)],
    )(agg, rin, w, b, fcw, fcb)


def _tc_combine_body(outs_ref, sp_ref, att_ref, hb_ref, res_ref):
    l0 = jnp.sum(att_ref[0, 0, :] * sp_ref[0, 0, :])
    l1 = jnp.sum(att_ref[0, 1, :] * sp_ref[0, 1, :])
    m = jnp.maximum(l0, l1)
    e0 = jnp.exp(l0 - m)
    e1 = jnp.exp(l1 - m)
    inv = 1.0 / (e0 + e1)
    res_ref[...] = (outs_ref[0] * (e0 * inv) + outs_ref[1] * (e1 * inv)
                    + hb_ref[0])


def _tc_combine(outs, sp, att, hb, blk=512):
    nb = NP // blk
    return pl.pallas_call(
        _tc_combine_body,
        grid=(nb,),
        in_specs=[
            pl.BlockSpec((NC, blk, D), lambda i: (0, i, 0)),
            pl.BlockSpec((1, NC, D), lambda i: (0, 0, 0)),
            pl.BlockSpec((1, NC, D), lambda i: (0, 0, 0)),
            pl.BlockSpec((1, D), lambda i: (0, 0)),
        ],
        out_specs=pl.BlockSpec((blk, D), lambda i: (i, 0)),
        out_shape=jax.ShapeDtypeStruct((N, D), jnp.float32),
    )(outs, sp, att, hb)


def _prep_idx(idx, pad_value, offset):
    padn = EP - E
    p = jnp.concatenate(
        [idx + offset, jnp.full((padn,), pad_value, jnp.int32)])
    return p.reshape(NS, CH, CL)


def kernel(inputs, edge_index0, edge_index1, W0, b0, W1, b1, att0, att1,
           fc_W, fc_b, h_bias):
    src0, dst0 = edge_index0[0], edge_index0[1]
    src1, dst1 = edge_index1[0], edge_index1[1]

    degsrc = jnp.stack([_prep_idx(src0, N + 8, 0), _prep_idx(src1, N + 8, 0)])
    dst = jnp.stack([_prep_idx(dst0, N, 0), _prep_idx(dst1, N, 0)])
    srcg = jnp.stack([_prep_idx(src0, 0, 0), _prep_idx(src1, 0, NP)])

    deg = _sc_degrees(degsrc, dst)

    h2, rin = _tc_scale(deg, inputs)

    srcg2 = jnp.stack([srcg, srcg + NC * NP])
    aggh = _sc_aggregate(h2.reshape(2 * NC * NP, DH), srcg2, dst)
    agg = jnp.concatenate([aggh[0], aggh[1]], axis=-1)

    w = jnp.stack([W0, W1])
    b = jnp.stack([b0, b1])[:, None, :]
    outs, sp = _tc_feat(agg, rin, w, b,
                        fc_W[None], fc_b[None])

    att = jnp.stack([att0[0], att1[0]])[None]
    res = _tc_combine(outs, sp[None], att, h_bias[None])
    return res


# speedup vs baseline: 4.8420x; 1.2425x over previous
import functools

import jax
import jax.numpy as jnp
from jax import lax
from jax.experimental import pallas as pl
from jax.experimental.pallas import tpu as pltpu
from jax.experimental.pallas import tpu_sc as plsc

N = 10000
E = 320000
D = 128
NC = 2
NS = 16
CL = 128
CH = (E // NS + CL - 1) // CL
EPT = CH * CL
EP = NS * EPT
NP = 10240
RPT = NP // NS

_mesh = plsc.VectorSubcoreMesh(core_axis_name="c", subcore_axis_name="s")


@functools.partial(
    pl.kernel,
    mesh=_mesh,
    out_type=jax.ShapeDtypeStruct((NC, 2, NP), jnp.float32),
    scratch_types=[
        pltpu.VMEM((CH, CL), jnp.int32),
        pltpu.VMEM((CL,), jnp.float32),
        pltpu.VMEM((RPT,), jnp.float32),
        pltpu.VMEM_SHARED((NP,), jnp.float32),
        pltpu.VMEM_SHARED((NP,), jnp.float32),
    ],
)
def _sc_degrees(degsrc_hbm, dst_hbm, out_hbm, idx_v, ones_v, stage_v,
                degs_sh, degd_sh):
    c = lax.axis_index("c")
    s = lax.axis_index("s")

    def _fill(i, _):
        stage_v[pl.ds(i * 16, 16)] = jnp.zeros((16,), jnp.float32)
        return 0
    lax.fori_loop(0, RPT // 16, _fill, 0)
    def _fill1(i, _):
        ones_v[pl.ds(i * 16, 16)] = jnp.ones((16,), jnp.float32)
        return 0
    lax.fori_loop(0, CL // 16, _fill1, 0)

    pltpu.sync_copy(stage_v, degs_sh.at[pl.ds(s * RPT, RPT)])
    pltpu.sync_copy(stage_v, degd_sh.at[pl.ds(s * RPT, RPT)])
    plsc.subcore_barrier()

    pltpu.sync_copy(degsrc_hbm.at[c, s], idx_v)
    def _cnt_s(j, _):
        pltpu.sync_copy(ones_v, degs_sh.at[idx_v.at[j]], add=True)
        return 0
    lax.fori_loop(0, CH, _cnt_s, 0)

    pltpu.sync_copy(dst_hbm.at[c, s], idx_v)
    def _cnt_d(j, _):
        pltpu.sync_copy(ones_v, degd_sh.at[idx_v.at[j]], add=True)
        return 0
    lax.fori_loop(0, CH, _cnt_d, 0)

    plsc.subcore_barrier()

    pltpu.sync_copy(degs_sh.at[pl.ds(s * RPT, RPT)], stage_v)
    pltpu.sync_copy(stage_v, out_hbm.at[c, 0, pl.ds(s * RPT, RPT)])
    pltpu.sync_copy(degd_sh.at[pl.ds(s * RPT, RPT)], stage_v)
    pltpu.sync_copy(stage_v, out_hbm.at[c, 1, pl.ds(s * RPT, RPT)])


DH = D // 2


@functools.partial(
    pl.kernel,
    mesh=_mesh,
    out_type=jax.ShapeDtypeStruct((2, NC, NP, DH), jnp.float32),
    compiler_params=pltpu.CompilerParams(use_tc_tiling_on_sc=False),
    scratch_types=[
        pltpu.VMEM((CH, CL), jnp.int32),
        pltpu.VMEM((CH, CL), jnp.int32),
        pltpu.VMEM((CL, DH), jnp.float32),
        pltpu.VMEM((CL, DH), jnp.float32),
        pltpu.VMEM((CL, DH), jnp.float32),
        pltpu.VMEM((CL, DH), jnp.float32),
        pltpu.VMEM_SHARED((NP, DH), jnp.float32),
        pltpu.SemaphoreType.DMA,
        pltpu.SemaphoreType.DMA,
        pltpu.SemaphoreType.DMA,
        pltpu.SemaphoreType.DMA,
        pltpu.SemaphoreType.DMA,
        pltpu.SemaphoreType.DMA,
        pltpu.SemaphoreType.DMA,
        pltpu.SemaphoreType.DMA,
    ],
)
def _sc_aggregate(h_hbm, src_hbm, dst_hbm, out_hbm, src_v, dst_v,
                  buf0, buf1, buf2, buf3, agg_sh,
                  gs0, gs1, gs2, gs3, ss0, ss1, ss2, ss3):
    c = lax.axis_index("c")
    s = lax.axis_index("s")

    pltpu.sync_copy(dst_hbm.at[c, s], dst_v)

    for p in range(2):
        def _fill(i, _):
            r = i // (DH // 16)
            k = i % (DH // 16)
            buf0[r, pl.ds(k * 16, 16)] = jnp.zeros((16,), jnp.float32)
            return 0
        lax.fori_loop(0, CL * (DH // 16), _fill, 0)
        for b in range(RPT // CL):
            pltpu.sync_copy(buf0, agg_sh.at[pl.ds(s * RPT + b * CL, CL), :])

        pltpu.sync_copy(src_hbm.at[p, c, s], src_v)
        plsc.subcore_barrier()

        def _step(jj, _):
            b = jj * 4
            g0 = pltpu.async_copy(h_hbm.at[src_v.at[b]], buf0, gs0)
            g1 = pltpu.async_copy(h_hbm.at[src_v.at[b + 1]], buf1, gs1)
            g2 = pltpu.async_copy(h_hbm.at[src_v.at[b + 2]], buf2, gs2)
            g3 = pltpu.async_copy(h_hbm.at[src_v.at[b + 3]], buf3, gs3)
            g0.wait()
            s0 = pltpu.async_copy(buf0, agg_sh.at[dst_v.at[b]], ss0, add=True)
            g1.wait()
            s1 = pltpu.async_copy(buf1, agg_sh.at[dst_v.at[b + 1]], ss1,
                                  add=True)
            g2.wait()
            s2 = pltpu.async_copy(buf2, agg_sh.at[dst_v.at[b + 2]], ss2,
                                  add=True)
            g3.wait()
            s3 = pltpu.async_copy(buf3, agg_sh.at[dst_v.at[b + 3]], ss3,
                                  add=True)
            s0.wait()
            s1.wait()
            s2.wait()
            s3.wait()
            return 0

        lax.fori_loop(0, CH // 4, _step, 0)
        for j in range((CH // 4) * 4, CH):
            pltpu.async_copy(h_hbm.at[src_v.at[j]], buf0, gs0).wait()
            pltpu.sync_copy(buf0, agg_sh.at[dst_v.at[j]], add=True)
        plsc.subcore_barrier()

        for b in range(RPT // CL):
            r0 = s * RPT + b * CL
            pltpu.sync_copy(agg_sh.at[pl.ds(r0, CL), :], buf0)
            pltpu.sync_copy(buf0, out_hbm.at[p, c, pl.ds(r0, CL), :])
        plsc.subcore_barrier()


def _tc_scale_body(deg_ref, x_ref, h_ref, rin_ref):
    g = pl.program_id(0)
    i = pl.program_id(1)
    blk = x_ref.shape[0]
    od = deg_ref[g, 0, pl.ds(i * blk, blk)]
    scale = lax.rsqrt(jnp.maximum(od, 1.0))
    h = x_ref[...] * scale[:, None]
    h_ref[0] = h[:, :D // 2]
    h_ref[1] = h[:, D // 2:]
    ind = deg_ref[g, 1, :]
    rin_ref[0, 0] = lax.rsqrt(jnp.maximum(ind, 1.0))


def _tc_scale(deg, x, blk=512):
    nb = NP // blk
    return pl.pallas_call(
        _tc_scale_body,
        grid=(NC, nb),
        in_specs=[
            pl.BlockSpec((NC, 2, NP), lambda g, i: (0, 0, 0)),
            pl.BlockSpec((blk, D), lambda g, i: (i, 0)),
        ],
        out_specs=[
            pl.BlockSpec((2, blk, D // 2), lambda g, i: (0, g * nb + i, 0)),
            pl.BlockSpec((1, 1, NP), lambda g, i: (g, 0, 0)),
        ],
        out_shape=[
            jax.ShapeDtypeStruct((2, NC * NP, D // 2), jnp.float32),
            jax.ShapeDtypeStruct((NC, 1, NP), jnp.float32),
        ],
    )(deg, x)


def _tc_feat_body(agg_ref, rin_ref, w_ref, b_ref, fcw_ref, fcb_ref,
                  out_ref, sp_ref, acc_ref):
    i = pl.program_id(0)
    blk = agg_ref.shape[1]

    @pl.when(i == 0)
    def _():
        acc_ref[...] = jnp.zeros_like(acc_ref)

    row0 = i * blk
    rows = row0 + lax.broadcasted_iota(jnp.int32, (blk, 1), 0)
    mask = rows < N

    for g in range(NC):
        scale = rin_ref[g, 0, pl.ds(row0, blk)][:, None]
        a = agg_ref[g] * scale
        out = jnp.dot(a, w_ref[g], preferred_element_type=jnp.float32)
        out = out + b_ref[g]
        out_ref[g] = out
        u = jnp.tanh(
            jax.lax.dot_general(out, fcw_ref[0],
                                (((1,), (1,)), ((), ())),
                                preferred_element_type=jnp.float32)
            + fcb_ref[0])
        u = jnp.where(mask, u, 0.0)
        acc_ref[g, :] += jnp.sum(u, axis=0)

    sp_ref[...] = acc_ref[...] * (1.0 / N)


def _tc_feat(agg, rin, w, b, fcw, fcb, blk=512):
    nb = NP // blk
    return pl.pallas_call(
        _tc_feat_body,
        grid=(nb,),
        in_specs=[
            pl.BlockSpec((NC, blk, D), lambda i: (0, i, 0)),
            pl.BlockSpec((NC, 1, NP), lambda i: (0, 0, 0)),
            pl.BlockSpec((NC, D, D), lambda i: (0, 0, 0)),
            pl.BlockSpec((NC, 1, D), lambda i: (0, 0, 0)),
            pl.BlockSpec((1, D, D), lambda i: (0, 0, 0)),
            pl.BlockSpec((1, D), lambda i: (0, 0)),
        ],
        out_specs=[
            pl.BlockSpec((NC, blk, D), lambda i: (0, i, 0)),
            pl.BlockSpec((NC, D), lambda i: (0, 0)),
        ],
        out_shape=[
            jax.ShapeDtypeStruct((NC, NP, D), jnp.float32),
            jax.ShapeDtypeStruct((NC, D), jnp.float32),
        ],
        scratch_shapes=[pltpu.VMEM((NC, D), jnp.float32)],
    )(agg, rin, w, b, fcw, fcb)


def _tc_combine_body(outs_ref, sp_ref, att_ref, hb_ref, res_ref):
    l0 = jnp.sum(att_ref[0, 0, :] * sp_ref[0, 0, :])
    l1 = jnp.sum(att_ref[0, 1, :] * sp_ref[0, 1, :])
    m = jnp.maximum(l0, l1)
    e0 = jnp.exp(l0 - m)
    e1 = jnp.exp(l1 - m)
    inv = 1.0 / (e0 + e1)
    res_ref[...] = (outs_ref[0] * (e0 * inv) + outs_ref[1] * (e1 * inv)
                    + hb_ref[0])


def _tc_combine(outs, sp, att, hb, blk=512):
    nb = NP // blk
    return pl.pallas_call(
        _tc_combine_body,
        grid=(nb,),
        in_specs=[
            pl.BlockSpec((NC, blk, D), lambda i: (0, i, 0)),
            pl.BlockSpec((1, NC, D), lambda i: (0, 0, 0)),
            pl.BlockSpec((1, NC, D), lambda i: (0, 0, 0)),
            pl.BlockSpec((1, D), lambda i: (0, 0)),
        ],
        out_specs=pl.BlockSpec((blk, D), lambda i: (i, 0)),
        out_shape=jax.ShapeDtypeStruct((N, D), jnp.float32),
    )(outs, sp, att, hb)


def _prep_idx(idx, pad_value, offset):
    padn = EP - E
    p = jnp.concatenate(
        [idx + offset, jnp.full((padn,), pad_value, jnp.int32)])
    return p.reshape(NS, CH, CL)


def kernel(inputs, edge_index0, edge_index1, W0, b0, W1, b1, att0, att1,
           fc_W, fc_b, h_bias):
    src0, dst0 = edge_index0[0], edge_index0[1]
    src1, dst1 = edge_index1[0], edge_index1[1]

    degsrc = jnp.stack([_prep_idx(src0, N + 8, 0), _prep_idx(src1, N + 8, 0)])
    dst = jnp.stack([_prep_idx(dst0, N, 0), _prep_idx(dst1, N, 0)])
    srcg = jnp.stack([_prep_idx(src0, 0, 0), _prep_idx(src1, 0, NP)])

    deg = _sc_degrees(degsrc, dst)

    h2, rin = _tc_scale(deg, inputs)

    srcg2 = jnp.stack([srcg, srcg + NC * NP])
    aggh = _sc_aggregate(h2.reshape(2 * NC * NP, DH), srcg2, dst)
    agg = jnp.concatenate([aggh[0], aggh[1]], axis=-1)

    w = jnp.stack([W0, W1])
    b = jnp.stack([b0, b1])[:, None, :]
    outs, sp = _tc_feat(agg, rin, w, b,
                        fc_W[None], fc_b[None])

    att = jnp.stack([att0[0], att1[0]])[None]
    res = _tc_combine(outs, sp[None], att, h_bias[None])
    return res


# speedup vs baseline: 5.1010x; 1.0535x over previous
import functools

import jax
import jax.numpy as jnp
from jax import lax
from jax.experimental import pallas as pl
from jax.experimental.pallas import tpu as pltpu
from jax.experimental.pallas import tpu_sc as plsc

N = 10000
E = 320000
D = 128
NC = 2
NS = 16
CL = 128
CH = (E // NS + CL - 1) // CL
EPT = CH * CL
EP = NS * EPT
NP = 10240
RPT = NP // NS

_mesh = plsc.VectorSubcoreMesh(core_axis_name="c", subcore_axis_name="s")


@functools.partial(
    pl.kernel,
    mesh=_mesh,
    out_type=jax.ShapeDtypeStruct((NC, 2, NP), jnp.float32),
    scratch_types=[
        pltpu.VMEM((CH, CL), jnp.int32),
        pltpu.VMEM((CL,), jnp.float32),
        pltpu.VMEM((RPT,), jnp.float32),
        pltpu.VMEM_SHARED((NP,), jnp.float32),
        pltpu.VMEM_SHARED((NP,), jnp.float32),
    ],
)
def _sc_degrees(degsrc_hbm, dst_hbm, out_hbm, idx_v, ones_v, stage_v,
                degs_sh, degd_sh):
    c = lax.axis_index("c")
    s = lax.axis_index("s")

    def _fill(i, _):
        stage_v[pl.ds(i * 16, 16)] = jnp.zeros((16,), jnp.float32)
        return 0
    lax.fori_loop(0, RPT // 16, _fill, 0)
    def _fill1(i, _):
        ones_v[pl.ds(i * 16, 16)] = jnp.ones((16,), jnp.float32)
        return 0
    lax.fori_loop(0, CL // 16, _fill1, 0)

    pltpu.sync_copy(stage_v, degs_sh.at[pl.ds(s * RPT, RPT)])
    pltpu.sync_copy(stage_v, degd_sh.at[pl.ds(s * RPT, RPT)])
    plsc.subcore_barrier()

    pltpu.sync_copy(degsrc_hbm.at[c, s], idx_v)
    def _cnt_s(j, _):
        pltpu.sync_copy(ones_v, degs_sh.at[idx_v.at[j]], add=True)
        return 0
    lax.fori_loop(0, CH, _cnt_s, 0)

    pltpu.sync_copy(dst_hbm.at[c, s], idx_v)
    def _cnt_d(j, _):
        pltpu.sync_copy(ones_v, degd_sh.at[idx_v.at[j]], add=True)
        return 0
    lax.fori_loop(0, CH, _cnt_d, 0)

    plsc.subcore_barrier()

    pltpu.sync_copy(degs_sh.at[pl.ds(s * RPT, RPT)], stage_v)
    pltpu.sync_copy(stage_v, out_hbm.at[c, 0, pl.ds(s * RPT, RPT)])
    pltpu.sync_copy(degd_sh.at[pl.ds(s * RPT, RPT)], stage_v)
    pltpu.sync_copy(stage_v, out_hbm.at[c, 1, pl.ds(s * RPT, RPT)])


DH = D // 2


@functools.partial(
    pl.kernel,
    mesh=_mesh,
    out_type=jax.ShapeDtypeStruct((2, NC, NP, DH), jnp.float32),
    compiler_params=pltpu.CompilerParams(use_tc_tiling_on_sc=False),
    scratch_types=[
        pltpu.VMEM((CH, CL), jnp.int32),
        pltpu.VMEM((CH, CL), jnp.int32),
        *([pltpu.VMEM((CL, DH), jnp.float32)] * 6),
        pltpu.VMEM_SHARED((NP, DH), jnp.float32),
        *([pltpu.SemaphoreType.DMA] * 12),
    ],
)
def _sc_aggregate(h_hbm, src_hbm, dst_hbm, out_hbm, src_v, dst_v, *rest):
    bufs = rest[:6]
    agg_sh = rest[6]
    gss = rest[7:13]
    sss = rest[13:19]
    buf0 = bufs[0]
    c = lax.axis_index("c")
    s = lax.axis_index("s")

    pltpu.sync_copy(dst_hbm.at[c, s], dst_v)

    for p in range(2):
        def _fill(i, _):
            r = i // (DH // 16)
            k = i % (DH // 16)
            buf0[r, pl.ds(k * 16, 16)] = jnp.zeros((16,), jnp.float32)
            return 0
        lax.fori_loop(0, CL * (DH // 16), _fill, 0)
        for b in range(RPT // CL):
            pltpu.sync_copy(buf0, agg_sh.at[pl.ds(s * RPT + b * CL, CL), :])

        pltpu.sync_copy(src_hbm.at[p, c, s], src_v)
        plsc.subcore_barrier()

        UN = 6

        def _step(jj, _):
            b = jj * UN
            gcopies = [
                pltpu.async_copy(h_hbm.at[src_v.at[b + k]], bufs[k], gss[k])
                for k in range(UN)
            ]
            scopies = []
            for k in range(UN):
                gcopies[k].wait()
                scopies.append(
                    pltpu.async_copy(bufs[k], agg_sh.at[dst_v.at[b + k]],
                                     sss[k], add=True))
            for sc in scopies:
                sc.wait()
            return 0

        lax.fori_loop(0, CH // UN, _step, 0)
        tail0 = (CH // UN) * UN
        tgs = [
            pltpu.async_copy(h_hbm.at[src_v.at[j]], bufs[j - tail0],
                             gss[j - tail0])
            for j in range(tail0, CH)
        ]
        tss = []
        for j in range(tail0, CH):
            tgs[j - tail0].wait()
            tss.append(
                pltpu.async_copy(bufs[j - tail0], agg_sh.at[dst_v.at[j]],
                                 sss[j - tail0], add=True))
        for sc in tss:
            sc.wait()
        plsc.subcore_barrier()

        for b in range(RPT // CL):
            r0 = s * RPT + b * CL
            pltpu.sync_copy(agg_sh.at[pl.ds(r0, CL), :], buf0)
            pltpu.sync_copy(buf0, out_hbm.at[p, c, pl.ds(r0, CL), :])
        plsc.subcore_barrier()


def _tc_scale_body(deg_ref, x_ref, h_ref, rin_ref):
    g = pl.program_id(0)
    i = pl.program_id(1)
    blk = x_ref.shape[0]
    od = deg_ref[g, 0, pl.ds(i * blk, blk)]
    scale = lax.rsqrt(jnp.maximum(od, 1.0))
    h = x_ref[...] * scale[:, None]
    h_ref[0] = h[:, :D // 2]
    h_ref[1] = h[:, D // 2:]
    ind = deg_ref[g, 1, :]
    rin_ref[0, 0] = lax.rsqrt(jnp.maximum(ind, 1.0))


def _tc_scale(deg, x, blk=512):
    nb = NP // blk
    return pl.pallas_call(
        _tc_scale_body,
        grid=(NC, nb),
        in_specs=[
            pl.BlockSpec((NC, 2, NP), lambda g, i: (0, 0, 0)),
            pl.BlockSpec((blk, D), lambda g, i: (i, 0)),
        ],
        out_specs=[
            pl.BlockSpec((2, blk, D // 2), lambda g, i: (0, g * nb + i, 0)),
            pl.BlockSpec((1, 1, NP), lambda g, i: (g, 0, 0)),
        ],
        out_shape=[
            jax.ShapeDtypeStruct((2, NC * NP, D // 2), jnp.float32),
            jax.ShapeDtypeStruct((NC, 1, NP), jnp.float32),
        ],
    )(deg, x)


def _tc_feat_body(agg_ref, rin_ref, w_ref, b_ref, fcw_ref, fcb_ref,
                  out_ref, sp_ref, acc_ref):
    i = pl.program_id(0)
    blk = agg_ref.shape[1]

    @pl.when(i == 0)
    def _():
        acc_ref[...] = jnp.zeros_like(acc_ref)

    row0 = i * blk
    rows = row0 + lax.broadcasted_iota(jnp.int32, (blk, 1), 0)
    mask = rows < N

    for g in range(NC):
        scale = rin_ref[g, 0, pl.ds(row0, blk)][:, None]
        a = agg_ref[g] * scale
        out = jnp.dot(a, w_ref[g], preferred_element_type=jnp.float32)
        out = out + b_ref[g]
        out_ref[g] = out
        u = jnp.tanh(
            jax.lax.dot_general(out, fcw_ref[0],
                                (((1,), (1,)), ((), ())),
                                preferred_element_type=jnp.float32)
            + fcb_ref[0])
        u = jnp.where(mask, u, 0.0)
        acc_ref[g, :] += jnp.sum(u, axis=0)

    sp_ref[...] = acc_ref[...] * (1.0 / N)


def _tc_feat(agg, rin, w, b, fcw, fcb, blk=512):
    nb = NP // blk
    return pl.pallas_call(
        _tc_feat_body,
        grid=(nb,),
        in_specs=[
            pl.BlockSpec((NC, blk, D), lambda i: (0, i, 0)),
            pl.BlockSpec((NC, 1, NP), lambda i: (0, 0, 0)),
            pl.BlockSpec((NC, D, D), lambda i: (0, 0, 0)),
            pl.BlockSpec((NC, 1, D), lambda i: (0, 0, 0)),
            pl.BlockSpec((1, D, D), lambda i: (0, 0, 0)),
            pl.BlockSpec((1, D), lambda i: (0, 0)),
        ],
        out_specs=[
            pl.BlockSpec((NC, blk, D), lambda i: (0, i, 0)),
            pl.BlockSpec((NC, D), lambda i: (0, 0)),
        ],
        out_shape=[
            jax.ShapeDtypeStruct((NC, NP, D), jnp.float32),
            jax.ShapeDtypeStruct((NC, D), jnp.float32),
        ],
        scratch_shapes=[pltpu.VMEM((NC, D), jnp.float32)],
    )(agg, rin, w, b, fcw, fcb)


def _tc_combine_body(outs_ref, sp_ref, att_ref, hb_ref, res_ref):
    l0 = jnp.sum(att_ref[0, 0, :] * sp_ref[0, 0, :])
    l1 = jnp.sum(att_ref[0, 1, :] * sp_ref[0, 1, :])
    m = jnp.maximum(l0, l1)
    e0 = jnp.exp(l0 - m)
    e1 = jnp.exp(l1 - m)
    inv = 1.0 / (e0 + e1)
    res_ref[...] = (outs_ref[0] * (e0 * inv) + outs_ref[1] * (e1 * inv)
                    + hb_ref[0])


def _tc_combine(outs, sp, att, hb, blk=512):
    nb = NP // blk
    return pl.pallas_call(
        _tc_combine_body,
        grid=(nb,),
        in_specs=[
            pl.BlockSpec((NC, blk, D), lambda i: (0, i, 0)),
            pl.BlockSpec((1, NC, D), lambda i: (0, 0, 0)),
            pl.BlockSpec((1, NC, D), lambda i: (0, 0, 0)),
            pl.BlockSpec((1, D), lambda i: (0, 0)),
        ],
        out_specs=pl.BlockSpec((blk, D), lambda i: (i, 0)),
        out_shape=jax.ShapeDtypeStruct((N, D), jnp.float32),
    )(outs, sp, att, hb)


def _prep_idx(idx, pad_value, offset):
    padn = EP - E
    p = jnp.concatenate(
        [idx + offset, jnp.full((padn,), pad_value, jnp.int32)])
    return p.reshape(NS, CH, CL)


def kernel(inputs, edge_index0, edge_index1, W0, b0, W1, b1, att0, att1,
           fc_W, fc_b, h_bias):
    src0, dst0 = edge_index0[0], edge_index0[1]
    src1, dst1 = edge_index1[0], edge_index1[1]

    degsrc = jnp.stack([_prep_idx(src0, N + 8, 0), _prep_idx(src1, N + 8, 0)])
    dst = jnp.stack([_prep_idx(dst0, N, 0), _prep_idx(dst1, N, 0)])
    srcg = jnp.stack([_prep_idx(src0, 0, 0), _prep_idx(src1, 0, NP)])

    deg = _sc_degrees(degsrc, dst)

    h2, rin = _tc_scale(deg, inputs)

    srcg2 = jnp.stack([srcg, srcg + NC * NP])
    aggh = _sc_aggregate(h2.reshape(2 * NC * NP, DH), srcg2, dst)
    agg = jnp.concatenate([aggh[0], aggh[1]], axis=-1)

    w = jnp.stack([W0, W1])
    b = jnp.stack([b0, b1])[:, None, :]
    outs, sp = _tc_feat(agg, rin, w, b,
                        fc_W[None], fc_b[None])

    att = jnp.stack([att0[0], att1[0]])[None]
    res = _tc_combine(outs, sp[None], att, h_bias[None])
    return res
